# baseline (speedup 1.0000x reference)
import jax
import jax.numpy as jnp
from jax import lax
from jax.experimental import pallas as pl
from jax.experimental.pallas import tpu as pltpu

P_MASK = 0.5
TIME_MASKING_PARA = 100
TIME_MASK_NUM = 1

BLK_B = 8


def _mask_bounds(B, T):
    key = jax.random.key(42)
    key, k_apply = jax.random.split(key)
    apply_mask = jax.random.uniform(k_apply) <= P_MASK
    starts_l, ends_l = [], []
    for _ in range(TIME_MASK_NUM):
        key, k_t, k_t0 = jax.random.split(key, 3)
        t = jax.random.randint(k_t, (), 0, TIME_MASKING_PARA + 1)
        t0s = jax.random.randint(k_t0, (B,), 0, T - TIME_MASKING_PARA)
        t_eff = jnp.where(apply_mask, t, 0)
        starts_l.append(t0s.astype(jnp.int32))
        ends_l.append((t0s + t_eff).astype(jnp.int32))
    return starts_l[0], ends_l[0]


def _body(m_ref, x_ref, o_ref):
    o_ref[...] = jnp.where(m_ref[...], jnp.float32(0.0), x_ref[...])


def kernel(mel_batch):
    B, F, T = mel_batch.shape
    starts, ends = _mask_bounds(B, T)
    idx = lax.broadcasted_iota(jnp.int32, (B, 1, T), 2)
    m = (idx >= starts[:, None, None]) & (idx < ends[:, None, None])
    out = pl.pallas_call(
        _body,
        grid=(B // BLK_B,),
        in_specs=[
            pl.BlockSpec((BLK_B, 1, T), lambda b: (b, 0, 0)),
            pl.BlockSpec((BLK_B, F, T), lambda b: (b, 0, 0)),
        ],
        out_specs=pl.BlockSpec((BLK_B, F, T), lambda b: (b, 0, 0)),
        out_shape=jax.ShapeDtypeStruct((B, F, T), jnp.float32),
    )(m, mel_batch)
    return out


# speedup vs baseline: 1.2910x; 1.2910x over previous
import jax
import jax.numpy as jnp
from jax import lax
from jax.experimental import pallas as pl
from jax.experimental.pallas import tpu as pltpu

P_MASK = 0.5
TIME_MASKING_PARA = 100
TIME_MASK_NUM = 1

NBUF = 4


def _mask_bounds(B, T):
    key = jax.random.key(42)
    key, k_apply = jax.random.split(key)
    apply_mask = jax.random.uniform(k_apply) <= P_MASK
    starts_l, ends_l = [], []
    for _ in range(TIME_MASK_NUM):
        key, k_t, k_t0 = jax.random.split(key, 3)
        t = jax.random.randint(k_t, (), 0, TIME_MASKING_PARA + 1)
        t0s = jax.random.randint(k_t0, (B,), 0, T - TIME_MASKING_PARA)
        t_eff = jnp.where(apply_mask, t, 0)
        starts_l.append(t0s.astype(jnp.int32))
        ends_l.append((t0s + t_eff).astype(jnp.int32))
    return starts_l[0], ends_l[0]


def _make_body(B, F, T):
    def body(m_ref, x_hbm, o_hbm, *rest):
        in_bufs = rest[0:NBUF]
        out_bufs = rest[NBUF:2 * NBUF]
        in_sems = rest[2 * NBUF:3 * NBUF]
        out_sems = rest[3 * NBUF:4 * NBUF]

        def in_copy(i, slot):
            return pltpu.make_async_copy(
                x_hbm.at[pl.ds(i * F, F), :], in_bufs[slot], in_sems[slot])

        def out_copy(i, slot):
            return pltpu.make_async_copy(
                out_bufs[slot], o_hbm.at[pl.ds(i * F, F), :], out_sems[slot])

        for i in range(NBUF):
            in_copy(i, i).start()
        for i in range(B):
            slot = i % NBUF
            in_copy(i, slot).wait()
            mrow = m_ref[i:i + 1, :]
            if i >= NBUF:
                out_copy(i - NBUF, slot).wait()
            out_bufs[slot][...] = in_bufs[slot][...] * mrow
            out_copy(i, slot).start()
            nxt = i + NBUF
            if nxt < B:
                in_copy(nxt, slot).start()
        for i in range(B - NBUF, B):
            out_copy(i, i % NBUF).wait()

    return body


def kernel(mel_batch):
    B, F, T = mel_batch.shape
    starts, ends = _mask_bounds(B, T)
    idx = lax.broadcasted_iota(jnp.int32, (B, T), 1)
    m = jnp.where((idx >= starts[:, None]) & (idx < ends[:, None]),
                  jnp.float32(0.0), jnp.float32(1.0))
    x2d = mel_batch.reshape(B * F, T)
    out = pl.pallas_call(
        _make_body(B, F, T),
        grid=(),
        in_specs=[
            pl.BlockSpec(memory_space=pltpu.VMEM),
            pl.BlockSpec(memory_space=pl.ANY),
        ],
        out_specs=pl.BlockSpec(memory_space=pl.ANY),
        out_shape=jax.ShapeDtypeStruct((B * F, T), jnp.float32),
        scratch_shapes=(
            [pltpu.VMEM((F, T), jnp.float32) for _ in range(2 * NBUF)]
            + [pltpu.SemaphoreType.DMA for _ in range(2 * NBUF)]
        ),
    )(m, x2d)
    return out.reshape(B, F, T)


# speedup vs baseline: 1.2935x; 1.0019x over previous
import jax
import jax.numpy as jnp
from jax import lax
from jax.experimental import pallas as pl
from jax.experimental.pallas import tpu as pltpu

P_MASK = 0.5
TIME_MASKING_PARA = 100
TIME_MASK_NUM = 1

NBUF = 8
ROWS = 64


def _mask_bounds(B, T):
    key = jax.random.key(42)
    key, k_apply = jax.random.split(key)
    apply_mask = jax.random.uniform(k_apply) <= P_MASK
    starts_l, ends_l = [], []
    for _ in range(TIME_MASK_NUM):
        key, k_t, k_t0 = jax.random.split(key, 3)
        t = jax.random.randint(k_t, (), 0, TIME_MASKING_PARA + 1)
        t0s = jax.random.randint(k_t0, (B,), 0, T - TIME_MASKING_PARA)
        t_eff = jnp.where(apply_mask, t, 0)
        starts_l.append(t0s.astype(jnp.int32))
        ends_l.append((t0s + t_eff).astype(jnp.int32))
    return starts_l[0], ends_l[0]


def _make_body(B, F, T):
    nchunks = B * F // ROWS
    per_sample = F // ROWS

    def body(m_ref, x_hbm, o_hbm, *rest):
        in_bufs = rest[0:NBUF]
        out_bufs = rest[NBUF:2 * NBUF]
        in_sems = rest[2 * NBUF:3 * NBUF]
        out_sems = rest[3 * NBUF:4 * NBUF]

        def in_copy(i, slot):
            return pltpu.make_async_copy(
                x_hbm.at[pl.ds(i * ROWS, ROWS), :], in_bufs[slot], in_sems[slot])

        def out_copy(i, slot):
            return pltpu.make_async_copy(
                out_bufs[slot], o_hbm.at[pl.ds(i * ROWS, ROWS), :], out_sems[slot])

        for i in range(NBUF):
            in_copy(i, i).start()
        for i in range(nchunks):
            slot = i % NBUF
            in_copy(i, slot).wait()
            s = i // per_sample
            mrow = m_ref[s:s + 1, :]
            if i >= NBUF:
                out_copy(i - NBUF, slot).wait()
            out_bufs[slot][...] = in_bufs[slot][...] * mrow
            out_copy(i, slot).start()
            nxt = i + NBUF
            if nxt < nchunks:
                in_copy(nxt, slot).start()
        for i in range(nchunks - NBUF, nchunks):
            out_copy(i, i % NBUF).wait()

    return body


def kernel(mel_batch):
    B, F, T = mel_batch.shape
    starts, ends = _mask_bounds(B, T)
    idx = lax.broadcasted_iota(jnp.int32, (B, T), 1)
    m = jnp.where((idx >= starts[:, None]) & (idx < ends[:, None]),
                  jnp.float32(0.0), jnp.float32(1.0))
    x2d = mel_batch.reshape(B * F, T)
    out = pl.pallas_call(
        _make_body(B, F, T),
        grid=(),
        in_specs=[
            pl.BlockSpec(memory_space=pltpu.VMEM),
            pl.BlockSpec(memory_space=pl.ANY),
        ],
        out_specs=pl.BlockSpec(memory_space=pl.ANY),
        out_shape=jax.ShapeDtypeStruct((B * F, T), jnp.float32),
        scratch_shapes=(
            [pltpu.VMEM((ROWS, T), jnp.float32) for _ in range(2 * NBUF)]
            + [pltpu.SemaphoreType.DMA for _ in range(2 * NBUF)]
        ),
    )(m, x2d)
    return out.reshape(B, F, T)
